# baseline (speedup 1.0000x reference)
import functools

import jax
import jax.numpy as jnp
from jax import lax
from jax.experimental import pallas as pl
from jax.experimental.pallas import tpu as pltpu
from jax.experimental.pallas import tpu_sc as plsc

_K = 512
_NITERS = 10
_D = 128

_BN = 1024

_NC = 2
_NS = 16
_NW = _NC * _NS
_CH = 128


def _assign_body(c_in_ref, x_ref, out_ref, cnt_ref, cs_ref, c2_ref, idf_ref):
    i = pl.program_id(0)

    @pl.when(i == 0)
    def _():
        c0 = c_in_ref[...]
        cs0 = 2.0 * c0
        cs_ref[...] = cs0
        c2_ref[...] = (0.25 * jnp.sum(cs0 * cs0, axis=1)).reshape(1, _K)
        idf_ref[...] = lax.broadcasted_iota(
            jnp.int32, (1, _K), 1).astype(jnp.float32)
        cnt_ref[...] = jnp.zeros_like(cnt_ref)

    cs = cs_ref[...]
    xb = x_ref[...]
    x2 = jnp.sum(xb * xb, axis=1, keepdims=True)
    prod2 = lax.dot_general(xb, cs, (((1,), (1,)), ((), ())),
                            preferred_element_type=jnp.float32)
    d = (x2 - prod2) + c2_ref[...]
    m = jnp.min(d, axis=1, keepdims=True)
    idf = idf_ref[...]
    chosen_f = jnp.min(jnp.where(d == m, idf, float(_K)), axis=1,
                       keepdims=True)
    out_ref[...] = chosen_f.astype(jnp.int32).reshape(_BN)

    onehot = (idf == chosen_f).astype(jnp.float32)
    cnt_ref[...] += jnp.sum(onehot, axis=0)


def _assign(x, c):
    n, d = x.shape
    nb = n // _BN
    return pl.pallas_call(
        _assign_body,
        grid=(nb,),
        in_specs=[
            pl.BlockSpec((_K, d), lambda i: (0, 0)),
            pl.BlockSpec((_BN, d), lambda i: (i, 0)),
        ],
        out_specs=[
            pl.BlockSpec((_BN,), lambda i: (i,)),
            pl.BlockSpec((_K,), lambda i: (0,)),
        ],
        out_shape=[
            jax.ShapeDtypeStruct((n,), jnp.int32),
            jax.ShapeDtypeStruct((_K,), jnp.float32),
        ],
        scratch_shapes=[
            pltpu.VMEM((_K, d), jnp.float32),
            pltpu.VMEM((1, _K), jnp.float32),
            pltpu.VMEM((1, _K), jnp.float32),
        ],
    )(c, x)


def _sc_update_body(x, clus, zeros, out, buf, idx, acc):
    cid = lax.axis_index("c")
    sid = lax.axis_index("s")
    wid = sid * _NC + cid
    rows_per_w = 65536 // _NW
    base = wid * rows_per_w
    rows_per_tile = _K // _NS

    pltpu.sync_copy(zeros.at[pl.ds(sid * rows_per_tile, rows_per_tile)],
                    acc.at[pl.ds(sid * rows_per_tile, rows_per_tile)])
    plsc.subcore_barrier()

    for g in range(rows_per_w // _CH):
        off = base + g * _CH
        pltpu.sync_copy(clus.at[pl.ds(off, _CH)], idx)
        pltpu.sync_copy(x.at[pl.ds(off, _CH)], buf)
        pltpu.sync_copy(buf, acc.at[idx], add=True)
    plsc.subcore_barrier()

    pltpu.sync_copy(acc.at[pl.ds(sid * rows_per_tile, rows_per_tile)],
                    out.at[cid, pl.ds(sid * rows_per_tile, rows_per_tile)])


@functools.cache
def _make_sc_update():
    return pl.kernel(
        _sc_update_body,
        out_type=jax.ShapeDtypeStruct((_NC, _K, _D), jnp.float32),
        mesh=plsc.VectorSubcoreMesh(core_axis_name="c", subcore_axis_name="s"),
        scratch_types=[
            pltpu.VMEM((_CH, _D), jnp.float32),
            pltpu.VMEM((_CH,), jnp.int32),
            pltpu.VMEM_SHARED((_K, _D), jnp.float32),
        ],
    )


def kernel(x):
    n, d = x.shape
    zeros = jnp.zeros((_K, _D), jnp.float32)

    def body(_, carry):
        centroids, _, _ = carry
        clusters, counts = _assign(x, centroids)
        part = _make_sc_update()(x, clusters, zeros)
        centroids = (part[0] + part[1]) / counts[:, None]
        return centroids, counts, clusters

    init = (x[:_K, :], jnp.ones((_K,), jnp.float32),
            jnp.zeros((n,), jnp.int32))
    centroids, counts, clusters = lax.fori_loop(0, _NITERS, body, init)
    return clusters.astype(jnp.int64), centroids, counts


# speedup vs baseline: 1.2273x; 1.2273x over previous
import functools

import jax
import jax.numpy as jnp
from jax import lax
from jax.experimental import pallas as pl
from jax.experimental.pallas import tpu as pltpu
from jax.experimental.pallas import tpu_sc as plsc

_K = 512
_NITERS = 10
_D = 128

_BN = 1024

_NC = 2
_NS = 16
_NW = _NC * _NS
_CH = 128


def _assign_body(x_ref, c_ref, out_ref, cnt_ref):
    xb = x_ref[...]
    c = c_ref[...]
    c2 = jnp.sum(c * c, axis=1)
    x2 = jnp.sum(xb * xb, axis=1, keepdims=True)
    prod = lax.dot_general(xb, c, (((1,), (1,)), ((), ())),
                           preferred_element_type=jnp.float32)
    d = x2 - 2.0 * prod + c2[None, :]
    m = jnp.min(d, axis=1, keepdims=True)
    ids = lax.broadcasted_iota(jnp.int32, d.shape, 1)
    chosen = jnp.min(jnp.where(d == m, ids, _K), axis=1)
    out_ref[...] = chosen.astype(jnp.int32)

    onehot = (ids == chosen[:, None]).astype(jnp.float32)
    blk_counts = jnp.sum(onehot, axis=0)

    @pl.when(pl.program_id(0) == 0)
    def _():
        cnt_ref[...] = jnp.zeros_like(cnt_ref)

    cnt_ref[...] += blk_counts


def _assign(x, centroids):
    n, d = x.shape
    nb = n // _BN
    return pl.pallas_call(
        _assign_body,
        grid=(nb,),
        in_specs=[
            pl.BlockSpec((_BN, d), lambda i: (i, 0)),
            pl.BlockSpec((_K, d), lambda i: (0, 0)),
        ],
        out_specs=[
            pl.BlockSpec((_BN,), lambda i: (i,)),
            pl.BlockSpec((_K,), lambda i: (0,)),
        ],
        out_shape=[
            jax.ShapeDtypeStruct((n,), jnp.int32),
            jax.ShapeDtypeStruct((_K,), jnp.float32),
        ],
    )(x, centroids)


def _sc_update_body(x, clus, zeros, out,
                    buf0, buf1, idx0, idx1, acc, sem0, sem1):
    cid = lax.axis_index("c")
    sid = lax.axis_index("s")
    wid = sid * _NC + cid
    rows_per_w = 65536 // _NW
    base = wid * rows_per_w
    rows_per_tile = _K // _NS
    nch = rows_per_w // _CH

    bufs = (buf0, buf1)
    idxs = (idx0, idx1)
    sems = (sem0, sem1)

    pending = [None, None]
    pending[0] = (
        pltpu.async_copy(clus.at[pl.ds(base, _CH)], idx0, sem0),
        pltpu.async_copy(x.at[pl.ds(base, _CH)], buf0, sem0),
    )

    pltpu.sync_copy(zeros.at[pl.ds(sid * rows_per_tile, rows_per_tile)],
                    acc.at[pl.ds(sid * rows_per_tile, rows_per_tile)])
    plsc.subcore_barrier()

    for g in range(nch):
        b = g & 1
        if g + 1 < nch:
            off = base + (g + 1) * _CH
            pending[1 - b] = (
                pltpu.async_copy(clus.at[pl.ds(off, _CH)], idxs[1 - b],
                                 sems[1 - b]),
                pltpu.async_copy(x.at[pl.ds(off, _CH)], bufs[1 - b],
                                 sems[1 - b]),
            )
        for cp in pending[b]:
            cp.wait()
        pltpu.sync_copy(bufs[b], acc.at[idxs[b]], add=True)
    plsc.subcore_barrier()

    pltpu.sync_copy(acc.at[pl.ds(sid * rows_per_tile, rows_per_tile)],
                    out.at[cid, pl.ds(sid * rows_per_tile, rows_per_tile)])


@functools.cache
def _make_sc_update():
    return pl.kernel(
        _sc_update_body,
        out_type=jax.ShapeDtypeStruct((_NC, _K, _D), jnp.float32),
        mesh=plsc.VectorSubcoreMesh(core_axis_name="c", subcore_axis_name="s"),
        scratch_types=[
            pltpu.VMEM((_CH, _D), jnp.float32),
            pltpu.VMEM((_CH, _D), jnp.float32),
            pltpu.VMEM((_CH,), jnp.int32),
            pltpu.VMEM((_CH,), jnp.int32),
            pltpu.VMEM_SHARED((_K, _D), jnp.float32),
            pltpu.SemaphoreType.DMA,
            pltpu.SemaphoreType.DMA,
        ],
    )


def kernel(x):
    n, d = x.shape
    zeros = jnp.zeros((_K, _D), jnp.float32)

    def body(_, carry):
        centroids, _, _ = carry
        clusters, counts = _assign(x, centroids)
        part = _make_sc_update()(x, clusters, zeros)
        sums = part[0] + part[1]
        centroids = sums / counts[:, None]
        return centroids, counts, clusters

    init = (x[:_K, :], jnp.ones((_K,), jnp.float32),
            jnp.zeros((n,), jnp.int32))
    centroids, counts, clusters = lax.fori_loop(0, _NITERS, body, init)
    return clusters.astype(jnp.int64), centroids, counts


# speedup vs baseline: 1.2970x; 1.0567x over previous
import functools

import jax
import jax.numpy as jnp
from jax import lax
from jax.experimental import pallas as pl
from jax.experimental.pallas import tpu as pltpu
from jax.experimental.pallas import tpu_sc as plsc

_K = 512
_NITERS = 10
_D = 128

_BN = 2048

_NC = 2
_NS = 16
_NW = _NC * _NS
_CH = 128


def _assign_body(x_ref, c_ref, out_ref, cnt_ref):
    xb = x_ref[...]
    c = c_ref[...]
    c2 = jnp.sum(c * c, axis=1)
    x2 = jnp.sum(xb * xb, axis=1, keepdims=True)
    prod = lax.dot_general(xb, c, (((1,), (1,)), ((), ())),
                           preferred_element_type=jnp.float32)
    d = x2 - 2.0 * prod + c2[None, :]
    m = jnp.min(d, axis=1, keepdims=True)
    ids = lax.broadcasted_iota(jnp.int32, d.shape, 1)
    chosen = jnp.min(jnp.where(d == m, ids, _K), axis=1)
    out_ref[...] = chosen.astype(jnp.int32)

    onehot = (ids == chosen[:, None]).astype(jnp.float32)
    blk_counts = jnp.sum(onehot, axis=0)

    @pl.when(pl.program_id(0) == 0)
    def _():
        cnt_ref[...] = jnp.zeros_like(cnt_ref)

    cnt_ref[...] += blk_counts


def _assign(x, centroids):
    n, d = x.shape
    nb = n // _BN
    return pl.pallas_call(
        _assign_body,
        grid=(nb,),
        in_specs=[
            pl.BlockSpec((_BN, d), lambda i: (i, 0)),
            pl.BlockSpec((_K, d), lambda i: (0, 0)),
        ],
        out_specs=[
            pl.BlockSpec((_BN,), lambda i: (i,)),
            pl.BlockSpec((_K,), lambda i: (0,)),
        ],
        out_shape=[
            jax.ShapeDtypeStruct((n,), jnp.int32),
            jax.ShapeDtypeStruct((_K,), jnp.float32),
        ],
    )(x, centroids)


def _sc_update_body(x, clus, zeros, out,
                    buf0, buf1, idx0, idx1, acc, sem0, sem1):
    cid = lax.axis_index("c")
    sid = lax.axis_index("s")
    wid = sid * _NC + cid
    rows_per_w = 65536 // _NW
    base = wid * rows_per_w
    rows_per_tile = _K // _NS
    nch = rows_per_w // _CH

    bufs = (buf0, buf1)
    idxs = (idx0, idx1)
    sems = (sem0, sem1)

    pending = [None, None]
    pending[0] = (
        pltpu.async_copy(clus.at[pl.ds(base, _CH)], idx0, sem0),
        pltpu.async_copy(x.at[pl.ds(base, _CH)], buf0, sem0),
    )

    pltpu.sync_copy(zeros.at[pl.ds(sid * rows_per_tile, rows_per_tile)],
                    acc.at[pl.ds(sid * rows_per_tile, rows_per_tile)])
    plsc.subcore_barrier()

    for g in range(nch):
        b = g & 1
        if g + 1 < nch:
            off = base + (g + 1) * _CH
            pending[1 - b] = (
                pltpu.async_copy(clus.at[pl.ds(off, _CH)], idxs[1 - b],
                                 sems[1 - b]),
                pltpu.async_copy(x.at[pl.ds(off, _CH)], bufs[1 - b],
                                 sems[1 - b]),
            )
        for cp in pending[b]:
            cp.wait()
        pltpu.sync_copy(bufs[b], acc.at[idxs[b]], add=True)
    plsc.subcore_barrier()

    pltpu.sync_copy(acc.at[pl.ds(sid * rows_per_tile, rows_per_tile)],
                    out.at[cid, pl.ds(sid * rows_per_tile, rows_per_tile)])


@functools.cache
def _make_sc_update():
    return pl.kernel(
        _sc_update_body,
        out_type=jax.ShapeDtypeStruct((_NC, _K, _D), jnp.float32),
        mesh=plsc.VectorSubcoreMesh(core_axis_name="c", subcore_axis_name="s"),
        scratch_types=[
            pltpu.VMEM((_CH, _D), jnp.float32),
            pltpu.VMEM((_CH, _D), jnp.float32),
            pltpu.VMEM((_CH,), jnp.int32),
            pltpu.VMEM((_CH,), jnp.int32),
            pltpu.VMEM_SHARED((_K, _D), jnp.float32),
            pltpu.SemaphoreType.DMA,
            pltpu.SemaphoreType.DMA,
        ],
    )


def kernel(x):
    n, d = x.shape
    zeros = jnp.zeros((_K, _D), jnp.float32)

    def body(_, carry):
        centroids, _, _ = carry
        clusters, counts = _assign(x, centroids)
        part = _make_sc_update()(x, clusters, zeros)
        sums = part[0] + part[1]
        centroids = sums / counts[:, None]
        return centroids, counts, clusters

    init = (x[:_K, :], jnp.ones((_K,), jnp.float32),
            jnp.zeros((n,), jnp.int32))
    centroids, counts, clusters = lax.fori_loop(0, _NITERS, body, init)
    return clusters.astype(jnp.int64), centroids, counts


# speedup vs baseline: 1.3216x; 1.0190x over previous
import functools

import jax
import jax.numpy as jnp
from jax import lax
from jax.experimental import pallas as pl
from jax.experimental.pallas import tpu as pltpu
from jax.experimental.pallas import tpu_sc as plsc

_K = 512
_NITERS = 10
_D = 128

_BN = 4096

_NC = 2
_NS = 16
_NW = _NC * _NS
_CH = 128


def _assign_body(x_ref, c_ref, out_ref, cnt_ref):
    xb = x_ref[...]
    c = c_ref[...]
    c2 = jnp.sum(c * c, axis=1)
    x2 = jnp.sum(xb * xb, axis=1, keepdims=True)
    prod = lax.dot_general(xb, c, (((1,), (1,)), ((), ())),
                           preferred_element_type=jnp.float32)
    d = x2 - 2.0 * prod + c2[None, :]
    m = jnp.min(d, axis=1, keepdims=True)
    ids = lax.broadcasted_iota(jnp.int32, d.shape, 1)
    chosen = jnp.min(jnp.where(d == m, ids, _K), axis=1)
    out_ref[...] = chosen.astype(jnp.int32)

    onehot = (ids == chosen[:, None]).astype(jnp.float32)
    blk_counts = jnp.sum(onehot, axis=0)

    @pl.when(pl.program_id(0) == 0)
    def _():
        cnt_ref[...] = jnp.zeros_like(cnt_ref)

    cnt_ref[...] += blk_counts


def _assign(x, centroids):
    n, d = x.shape
    nb = n // _BN
    return pl.pallas_call(
        _assign_body,
        grid=(nb,),
        in_specs=[
            pl.BlockSpec((_BN, d), lambda i: (i, 0)),
            pl.BlockSpec((_K, d), lambda i: (0, 0)),
        ],
        out_specs=[
            pl.BlockSpec((_BN,), lambda i: (i,)),
            pl.BlockSpec((_K,), lambda i: (0,)),
        ],
        out_shape=[
            jax.ShapeDtypeStruct((n,), jnp.int32),
            jax.ShapeDtypeStruct((_K,), jnp.float32),
        ],
    )(x, centroids)


def _sc_update_body(x, clus, zeros, out,
                    buf0, buf1, idx0, idx1, acc, sem0, sem1):
    cid = lax.axis_index("c")
    sid = lax.axis_index("s")
    wid = sid * _NC + cid
    rows_per_w = 65536 // _NW
    base = wid * rows_per_w
    rows_per_tile = _K // _NS
    nch = rows_per_w // _CH

    bufs = (buf0, buf1)
    idxs = (idx0, idx1)
    sems = (sem0, sem1)

    pending = [None, None]
    pending[0] = (
        pltpu.async_copy(clus.at[pl.ds(base, _CH)], idx0, sem0),
        pltpu.async_copy(x.at[pl.ds(base, _CH)], buf0, sem0),
    )

    pltpu.sync_copy(zeros.at[pl.ds(sid * rows_per_tile, rows_per_tile)],
                    acc.at[pl.ds(sid * rows_per_tile, rows_per_tile)])
    plsc.subcore_barrier()

    for g in range(nch):
        b = g & 1
        if g + 1 < nch:
            off = base + (g + 1) * _CH
            pending[1 - b] = (
                pltpu.async_copy(clus.at[pl.ds(off, _CH)], idxs[1 - b],
                                 sems[1 - b]),
                pltpu.async_copy(x.at[pl.ds(off, _CH)], bufs[1 - b],
                                 sems[1 - b]),
            )
        for cp in pending[b]:
            cp.wait()
        pltpu.sync_copy(bufs[b], acc.at[idxs[b]], add=True)
    plsc.subcore_barrier()

    pltpu.sync_copy(acc.at[pl.ds(sid * rows_per_tile, rows_per_tile)],
                    out.at[cid, pl.ds(sid * rows_per_tile, rows_per_tile)])


@functools.cache
def _make_sc_update():
    return pl.kernel(
        _sc_update_body,
        out_type=jax.ShapeDtypeStruct((_NC, _K, _D), jnp.float32),
        mesh=plsc.VectorSubcoreMesh(core_axis_name="c", subcore_axis_name="s"),
        scratch_types=[
            pltpu.VMEM((_CH, _D), jnp.float32),
            pltpu.VMEM((_CH, _D), jnp.float32),
            pltpu.VMEM((_CH,), jnp.int32),
            pltpu.VMEM((_CH,), jnp.int32),
            pltpu.VMEM_SHARED((_K, _D), jnp.float32),
            pltpu.SemaphoreType.DMA,
            pltpu.SemaphoreType.DMA,
        ],
    )


def kernel(x):
    n, d = x.shape
    zeros = jnp.zeros((_K, _D), jnp.float32)

    def body(_, carry):
        centroids, _, _ = carry
        clusters, counts = _assign(x, centroids)
        part = _make_sc_update()(x, clusters, zeros)
        sums = part[0] + part[1]
        centroids = sums / counts[:, None]
        return centroids, counts, clusters

    init = (x[:_K, :], jnp.ones((_K,), jnp.float32),
            jnp.zeros((n,), jnp.int32))
    centroids, counts, clusters = lax.fori_loop(0, _NITERS, body, init)
    return clusters.astype(jnp.int64), centroids, counts
